# Initial kernel scaffold; baseline (speedup 1.0000x reference)
import jax
import jax.numpy as jnp
from jax.experimental import pallas as pl


def kernel(x, norm1_g, norm1_b, norm2_g, norm2_b, qkv_w, out_w, out_b, thr, gating_w, e_w1, e_b1, e_w2, e_b2):
    raise NotImplementedError("write your pallas kernel here")


# speedup vs baseline: 2.4470x; 2.4470x over previous
import functools
import jax
import jax.numpy as jnp
from jax.experimental import pallas as pl
from jax.experimental.pallas import tpu as pltpu

D = 768
H = 12
E = 8
TOPK = 2
INNER = 3072
N = 2048
DH = D // H

BN = 256
BS = 256
PMAX = TOPK * N + E * BS
NB = PMAX // BS


def _ln(x, g, b):
    mu = jnp.mean(x, axis=-1, keepdims=True)
    var = jnp.mean((x - mu) ** 2, axis=-1, keepdims=True)
    return (x - mu) * jax.lax.rsqrt(var + 1e-5) * g + b


def _bf16_dot(a, b):
    return jax.lax.dot(a.astype(jnp.bfloat16), b.astype(jnp.bfloat16),
                       preferred_element_type=jnp.float32)


def _qkv_kernel(x_ref, g_ref, b_ref, w_ref, o_ref):
    h = _ln(x_ref[...], g_ref[...], b_ref[...])
    o_ref[...] = _bf16_dot(h, w_ref[...])


def _qkv_call(x, g, b, w):
    return pl.pallas_call(
        _qkv_kernel,
        grid=(N // BN,),
        in_specs=[
            pl.BlockSpec((BN, D), lambda i: (i, 0)),
            pl.BlockSpec((1, D), lambda i: (0, 0)),
            pl.BlockSpec((1, D), lambda i: (0, 0)),
            pl.BlockSpec((D, 3 * D), lambda i: (0, 0)),
        ],
        out_specs=pl.BlockSpec((BN, 3 * D), lambda i: (i, 0)),
        out_shape=jax.ShapeDtypeStruct((N, 3 * D), jnp.float32),
    )(x, g, b, w)


def _attn_kernel(thr_ref, q_ref, k_ref, v_ref, o_ref):
    scale = DH ** -0.5
    q = q_ref[0, 0].astype(jnp.bfloat16)
    k = k_ref[0, 0].astype(jnp.bfloat16)
    s = jax.lax.dot_general(q, k, (((1,), (1,)), ((), ())),
                            preferred_element_type=jnp.float32) * scale
    thr = thr_ref[0]
    s = jnp.where(s < thr, jnp.float32(-1e9), s)
    m = jnp.max(s, axis=1, keepdims=True)
    p = jnp.exp(s - m)
    l = jnp.sum(p, axis=1, keepdims=True)
    o = jax.lax.dot(p.astype(jnp.bfloat16), v_ref[0, 0].astype(jnp.bfloat16),
                    preferred_element_type=jnp.float32)
    o_ref[0] = o / l


def _attn_call(qkv3, thr):
    BQ = 256
    return pl.pallas_call(
        _attn_kernel,
        grid=(H, N // BQ),
        in_specs=[
            pl.BlockSpec(memory_space=pltpu.SMEM),
            pl.BlockSpec((1, 1, BQ, DH), lambda h, j: (0, h, j, 0)),
            pl.BlockSpec((1, 1, N, DH), lambda h, j: (1, h, 0, 0)),
            pl.BlockSpec((1, 1, N, DH), lambda h, j: (2, h, 0, 0)),
        ],
        out_specs=pl.BlockSpec((1, BQ, DH), lambda h, j: (h, j, 0)),
        out_shape=jax.ShapeDtypeStruct((H, N, DH), jnp.float32),
    )(thr.reshape(1), qkv3, qkv3, qkv3)


def _proj_kernel(x_ref, a_ref, ow_ref, ob_ref, g2_ref, b2_ref, gw_ref,
                 x2_ref, h2_ref, lg_ref):
    x2 = x_ref[...] + _bf16_dot(a_ref[...], ow_ref[...]) + ob_ref[...]
    x2_ref[...] = x2
    h2 = _ln(x2, g2_ref[...], b2_ref[...])
    h2_ref[...] = h2
    lg_ref[...] = _bf16_dot(h2, gw_ref[...])


def _proj_call(x, attn_out, ow, ob, g2, b2, gw_pad):
    return pl.pallas_call(
        _proj_kernel,
        grid=(N // BN,),
        in_specs=[
            pl.BlockSpec((BN, D), lambda i: (i, 0)),
            pl.BlockSpec((BN, D), lambda i: (i, 0)),
            pl.BlockSpec((D, D), lambda i: (0, 0)),
            pl.BlockSpec((1, D), lambda i: (0, 0)),
            pl.BlockSpec((1, D), lambda i: (0, 0)),
            pl.BlockSpec((1, D), lambda i: (0, 0)),
            pl.BlockSpec((D, 128), lambda i: (0, 0)),
        ],
        out_specs=[
            pl.BlockSpec((BN, D), lambda i: (i, 0)),
            pl.BlockSpec((BN, D), lambda i: (i, 0)),
            pl.BlockSpec((BN, 128), lambda i: (i, 0)),
        ],
        out_shape=[
            jax.ShapeDtypeStruct((N, D), jnp.float32),
            jax.ShapeDtypeStruct((N, D), jnp.float32),
            jax.ShapeDtypeStruct((N, 128), jnp.float32),
        ],
    )(x, attn_out, ow, ob, g2, b2, gw_pad)


def _ffn_kernel(be_ref, act_ref, tok_ref, h2_ref, w1_ref, b1_ref, w2_ref,
                b2_ref, eo_ref, xs_ref):
    i = pl.program_id(0)

    @pl.when(act_ref[i] == 1)
    def _():
        def gather_row(r, _):
            t = tok_ref[i * BS + r]
            xs_ref[pl.ds(r, 1), :] = h2_ref[pl.ds(t, 1), :]
            return 0
        jax.lax.fori_loop(0, BS, gather_row, 0)
        h = _bf16_dot(xs_ref[...], w1_ref[0]) + b1_ref[0]
        h = h * 0.5 * (1.0 + jax.lax.erf(h * (2.0 ** -0.5)))
        o = _bf16_dot(h, w2_ref[0]) + b2_ref[0]
        eo_ref[...] = o


def _ffn_call(block_expert, block_active, tok_pad, h2, w1, b1, w2, b2):
    grid_spec = pltpu.PrefetchScalarGridSpec(
        num_scalar_prefetch=3,
        grid=(NB,),
        in_specs=[
            pl.BlockSpec((N, D), lambda i, be, act, tok: (0, 0)),
            pl.BlockSpec((1, D, INNER), lambda i, be, act, tok: (be[i], 0, 0)),
            pl.BlockSpec((1, 1, INNER), lambda i, be, act, tok: (be[i], 0, 0)),
            pl.BlockSpec((1, INNER, D), lambda i, be, act, tok: (be[i], 0, 0)),
            pl.BlockSpec((1, 1, D), lambda i, be, act, tok: (be[i], 0, 0)),
        ],
        out_specs=pl.BlockSpec((BS, D), lambda i, be, act, tok: (i, 0)),
        scratch_shapes=[pltpu.VMEM((BS, D), jnp.float32)],
    )
    return pl.pallas_call(
        _ffn_kernel,
        grid_spec=grid_spec,
        out_shape=jax.ShapeDtypeStruct((PMAX, D), jnp.float32),
    )(block_expert, block_active, tok_pad, h2, w1, b1, w2, b2)


def _combine_kernel(pos0_ref, pos1_ref, w0_ref, w1_ref, x2_ref, eo_ref,
                    o_ref):
    j = pl.program_id(0)

    def row(r, _):
        t = j * BN + r
        g0 = eo_ref[pl.ds(pos0_ref[t], 1), :]
        g1 = eo_ref[pl.ds(pos1_ref[t], 1), :]
        o_ref[pl.ds(r, 1), :] = (x2_ref[pl.ds(r, 1), :]
                                 + w0_ref[t] * g0 + w1_ref[t] * g1)
        return 0
    jax.lax.fori_loop(0, BN, row, 0)


def _combine_call(pos0, pos1, w0, w1, x2, eo):
    grid_spec = pltpu.PrefetchScalarGridSpec(
        num_scalar_prefetch=4,
        grid=(N // BN,),
        in_specs=[
            pl.BlockSpec((BN, D), lambda j, p0, p1, w0, w1: (j, 0)),
            pl.BlockSpec((PMAX, D), lambda j, p0, p1, w0, w1: (0, 0)),
        ],
        out_specs=pl.BlockSpec((BN, D), lambda j, p0, p1, w0, w1: (j, 0)),
        scratch_shapes=[],
    )
    return pl.pallas_call(
        _combine_kernel,
        grid_spec=grid_spec,
        out_shape=jax.ShapeDtypeStruct((N, D), jnp.float32),
    )(pos0, pos1, w0, w1, x2, eo)


def kernel(x, norm1_g, norm1_b, norm2_g, norm2_b, qkv_w, out_w, out_b, thr,
           gating_w, e_w1, e_b1, e_w2, e_b2):
    b, n, d = x.shape
    xf = x.reshape(n, d)

    qkv = _qkv_call(xf, norm1_g.reshape(1, D), norm1_b.reshape(1, D), qkv_w)
    qkv3 = qkv.reshape(N, 3, H, DH).transpose(1, 2, 0, 3)
    attn_h = _attn_call(qkv3, thr)
    attn_out = attn_h.transpose(1, 0, 2).reshape(N, D)
    gw_pad = jnp.pad(gating_w, ((0, 0), (0, 128 - E)))
    x2, h2, lg_pad = _proj_call(xf, attn_out, out_w, out_b.reshape(1, D),
                                norm2_g.reshape(1, D), norm2_b.reshape(1, D),
                                gw_pad)
    logits = lg_pad[:, :E]

    probs = jax.nn.softmax(logits, axis=-1)
    aux = E * jnp.sum(jnp.mean(probs, axis=0) * jnp.sum(probs, axis=0))

    tw, ti = jax.lax.top_k(logits, TOPK)
    tw = jax.nn.softmax(tw, axis=-1)
    eflat = ti.reshape(-1).astype(jnp.int32)
    sort_idx = jnp.argsort(eflat)
    sorted_e = eflat[sort_idx]
    sorted_t = (sort_idx // TOPK).astype(jnp.int32)
    counts = jnp.sum(jax.nn.one_hot(eflat, E, dtype=jnp.int32), axis=0)
    offs = jnp.concatenate([jnp.zeros(1, jnp.int32), jnp.cumsum(counts)[:-1]])
    pcounts = ((counts + BS - 1) // BS) * BS
    poffs = jnp.concatenate([jnp.zeros(1, jnp.int32), jnp.cumsum(pcounts)[:-1]])
    ptotal = jnp.sum(pcounts)
    i_arr = jnp.arange(TOPK * N, dtype=jnp.int32)
    ppos = poffs[sorted_e] + (i_arr - offs[sorted_e])
    tok_pad = jnp.zeros(PMAX, jnp.int32).at[ppos].set(sorted_t)
    pos_of_asg = jnp.zeros(TOPK * N, jnp.int32).at[sort_idx].set(ppos)
    pos0 = pos_of_asg[0::TOPK]
    pos1 = pos_of_asg[1::TOPK]
    blk_start = jnp.arange(NB, dtype=jnp.int32) * BS
    block_expert = jnp.clip(
        jnp.searchsorted(poffs, blk_start, side='right').astype(jnp.int32) - 1,
        0, E - 1)
    block_active = (blk_start < ptotal).astype(jnp.int32)

    eo = _ffn_call(block_expert, block_active, tok_pad, h2, e_w1,
                   e_b1.reshape(E, 1, INNER), e_w2, e_b2.reshape(E, 1, D))
    out = _combine_call(pos0, pos1, tw[:, 0], tw[:, 1], x2, eo)
    return (out.reshape(b, n, d), aux)
